# baseline (speedup 1.0000x reference)
import functools

import jax
import jax.numpy as jnp
from jax import lax
from jax.experimental import pallas as pl
from jax.experimental.pallas import tpu as pltpu
from jax.experimental.pallas import tpu_sc as plsc

EMBED = 64
BATCH = 16384
LANES = 16
TILE_R = 2
SHIFT = 1
MASK = TILE_R - 1
NUM_CORES = 2
NUM_SUBCORES = 16
NW = NUM_CORES * NUM_SUBCORES
CHUNK = 64
NCH = BATCH // (NW * CHUNK)

_MESH = plsc.VectorSubcoreMesh(
    core_axis_name="c", subcore_axis_name="s",
    num_cores=NUM_CORES, num_subcores=NUM_SUBCORES)


@functools.partial(
    pl.kernel,
    out_type=jax.ShapeDtypeStruct((NW, NCH, CHUNK, EMBED), jnp.float32),
    mesh=_MESH,
    scratch_types=[
        pltpu.VMEM((NCH, CHUNK), jnp.int32),
        pltpu.VMEM((2, CHUNK, TILE_R, EMBED), jnp.float32),
        pltpu.VMEM((2, CHUNK, EMBED), jnp.float32),
        pltpu.VMEM((2, CHUNK, EMBED), jnp.float32),
        [pltpu.SemaphoreType.DMA] * 2,
        [pltpu.SemaphoreType.DMA] * 2,
        [pltpu.SemaphoreType.DMA] * 2,
    ],
)
def _sc_embed(x_hbm, idx_hbm, table_hbm, out_hbm,
              idx_v, gath_v, x_v, out_v, gsems, xsems, osems):
    wid = lax.axis_index("s") * NUM_CORES + lax.axis_index("c")

    pltpu.sync_copy(idx_hbm.at[wid], idx_v)

    def issue_chunk(c, b):
        pltpu.async_copy(x_hbm.at[wid].at[c], x_v.at[b], xsems[b])
        for g in range(CHUNK // LANES):
            tvec = lax.shift_right_logical(
                idx_v[c, pl.ds(g * LANES, LANES)], SHIFT)
            for l in range(LANES):
                pltpu.async_copy(
                    table_hbm.at[tvec[l]],
                    gath_v.at[b].at[g * LANES + l],
                    gsems[b])

    def process_chunk(c, b):
        pltpu.make_async_copy(
            table_hbm.at[pl.ds(0, CHUNK)], gath_v.at[b], gsems[b]).wait()
        pltpu.make_async_copy(
            x_hbm.at[wid].at[0], x_v.at[b], xsems[b]).wait()
        @pl.when(c >= 2)
        def _():
            pltpu.make_async_copy(
                out_v.at[b], out_hbm.at[wid].at[0], osems[b]).wait()

        for g in range(CHUNK // LANES):
            svec = lax.bitwise_and(
                idx_v[c, pl.ds(g * LANES, LANES)], MASK)
            for l in range(LANES):
                s = svec[l]
                j = g * LANES + l
                for d in range(EMBED // LANES):
                    dsl = pl.ds(d * LANES, LANES)
                    out_v[b, j, dsl] = gath_v[b, j, s, dsl] * x_v[b, j, dsl]

        pltpu.async_copy(out_v.at[b], out_hbm.at[wid].at[c], osems[b])

    issue_chunk(0, 0)

    def pair_body(i, carry):
        c0 = i * 2
        issue_chunk(c0 + 1, 1)
        process_chunk(c0, 0)

        @pl.when(c0 + 2 < NCH)
        def _():
            issue_chunk(c0 + 2, 0)

        process_chunk(c0 + 1, 1)
        return carry

    lax.fori_loop(0, NCH // 2, pair_body, 0)

    for b in range(2):
        pltpu.make_async_copy(
            out_v.at[b], out_hbm.at[wid].at[0], osems[b]).wait()


def kernel(x, id, table):
    idx = id.astype(jnp.int32).reshape(NW, NCH, CHUNK)
    x_r = x.reshape(NW, NCH, CHUNK, EMBED)
    table_t = table.reshape(table.shape[0] // TILE_R, TILE_R, EMBED)
    out = _sc_embed(x_r, idx, table_t)
    return out.reshape(BATCH, EMBED)


# speedup vs baseline: 1.0079x; 1.0079x over previous
import functools

import jax
import jax.numpy as jnp
from jax import lax
from jax.experimental import pallas as pl
from jax.experimental.pallas import tpu as pltpu
from jax.experimental.pallas import tpu_sc as plsc

EMBED = 64
BATCH = 16384
LANES = 16
TILE_R = 2
SHIFT = 1
MASK = TILE_R - 1
NUM_CORES = 2
NUM_SUBCORES = 16
NW = NUM_CORES * NUM_SUBCORES
CHUNK = 16
NCH = BATCH // (NW * CHUNK)

_MESH = plsc.VectorSubcoreMesh(
    core_axis_name="c", subcore_axis_name="s",
    num_cores=NUM_CORES, num_subcores=NUM_SUBCORES)


@functools.partial(
    pl.kernel,
    out_type=jax.ShapeDtypeStruct((NW, NCH, CHUNK, EMBED), jnp.float32),
    mesh=_MESH,
    scratch_types=[
        pltpu.VMEM((NCH, CHUNK), jnp.int32),
        pltpu.VMEM((2, CHUNK, TILE_R, EMBED), jnp.float32),
        pltpu.VMEM((2, CHUNK, EMBED), jnp.float32),
        pltpu.VMEM((2, CHUNK, EMBED), jnp.float32),
        [pltpu.SemaphoreType.DMA] * 2,
        [pltpu.SemaphoreType.DMA] * 2,
        [pltpu.SemaphoreType.DMA] * 2,
    ],
)
def _sc_embed(x_hbm, idx_hbm, table_hbm, out_hbm,
              idx_v, gath_v, x_v, out_v, gsems, xsems, osems):
    wid = lax.axis_index("s") * NUM_CORES + lax.axis_index("c")

    pltpu.sync_copy(idx_hbm.at[wid], idx_v)

    def issue_chunk(c, b):
        pltpu.async_copy(x_hbm.at[wid].at[c], x_v.at[b], xsems[b])
        for g in range(CHUNK // LANES):
            tvec = lax.shift_right_logical(
                idx_v[c, pl.ds(g * LANES, LANES)], SHIFT)
            for l in range(LANES):
                pltpu.async_copy(
                    table_hbm.at[tvec[l]],
                    gath_v.at[b].at[g * LANES + l],
                    gsems[b])

    def process_chunk(c, b):
        pltpu.make_async_copy(
            table_hbm.at[pl.ds(0, CHUNK)], gath_v.at[b], gsems[b]).wait()
        pltpu.make_async_copy(
            x_hbm.at[wid].at[0], x_v.at[b], xsems[b]).wait()
        @pl.when(c >= 2)
        def _():
            pltpu.make_async_copy(
                out_v.at[b], out_hbm.at[wid].at[0], osems[b]).wait()

        for g in range(CHUNK // LANES):
            svec = lax.bitwise_and(
                idx_v[c, pl.ds(g * LANES, LANES)], MASK)
            for l in range(LANES):
                s = svec[l]
                j = g * LANES + l
                for d in range(EMBED // LANES):
                    dsl = pl.ds(d * LANES, LANES)
                    out_v[b, j, dsl] = gath_v[b, j, s, dsl] * x_v[b, j, dsl]

        pltpu.async_copy(out_v.at[b], out_hbm.at[wid].at[c], osems[b])

    issue_chunk(0, 0)

    def pair_body(i, carry):
        c0 = i * 2
        issue_chunk(c0 + 1, 1)
        process_chunk(c0, 0)

        @pl.when(c0 + 2 < NCH)
        def _():
            issue_chunk(c0 + 2, 0)

        process_chunk(c0 + 1, 1)
        return carry

    lax.fori_loop(0, NCH // 2, pair_body, 0)

    for b in range(2):
        pltpu.make_async_copy(
            out_v.at[b], out_hbm.at[wid].at[0], osems[b]).wait()


def kernel(x, id, table):
    idx = id.astype(jnp.int32).reshape(NW, NCH, CHUNK)
    x_r = x.reshape(NW, NCH, CHUNK, EMBED)
    table_t = table.reshape(table.shape[0] // TILE_R, TILE_R, EMBED)
    out = _sc_embed(x_r, idx, table_t)
    return out.reshape(BATCH, EMBED)


# speedup vs baseline: 1.0226x; 1.0146x over previous
import functools

import jax
import jax.numpy as jnp
from jax import lax
from jax.experimental import pallas as pl
from jax.experimental.pallas import tpu as pltpu
from jax.experimental.pallas import tpu_sc as plsc

EMBED = 64
BATCH = 16384
LANES = 16
TILE_R = 2
SHIFT = 1
MASK = TILE_R - 1
NUM_CORES = 2
NUM_SUBCORES = 16
NW = NUM_CORES * NUM_SUBCORES
CHUNK = 32
NCH = BATCH // (NW * CHUNK)

_MESH = plsc.VectorSubcoreMesh(
    core_axis_name="c", subcore_axis_name="s",
    num_cores=NUM_CORES, num_subcores=NUM_SUBCORES)


@functools.partial(
    pl.kernel,
    out_type=jax.ShapeDtypeStruct((NW, NCH, CHUNK, EMBED), jnp.float32),
    mesh=_MESH,
    scratch_types=[
        pltpu.VMEM((NCH, CHUNK), jnp.int32),
        pltpu.VMEM((2, CHUNK, TILE_R, EMBED), jnp.float32),
        pltpu.VMEM((2, CHUNK, EMBED), jnp.float32),
        pltpu.VMEM((2, CHUNK, EMBED), jnp.float32),
        [pltpu.SemaphoreType.DMA] * 2,
        [pltpu.SemaphoreType.DMA] * 2,
        [pltpu.SemaphoreType.DMA] * 2,
    ],
)
def _sc_embed(x_hbm, idx_hbm, table_hbm, out_hbm,
              idx_v, gath_v, x_v, out_v, gsems, xsems, osems):
    wid = lax.axis_index("s") * NUM_CORES + lax.axis_index("c")

    pltpu.sync_copy(idx_hbm.at[wid], idx_v)

    def issue_chunk(c, b):
        pltpu.async_copy(x_hbm.at[wid].at[c], x_v.at[b], xsems[b])
        for g in range(CHUNK // LANES):
            tvec = lax.shift_right_logical(
                idx_v[c, pl.ds(g * LANES, LANES)], SHIFT)
            for l in range(LANES):
                pltpu.async_copy(
                    table_hbm.at[tvec[l]],
                    gath_v.at[b].at[g * LANES + l],
                    gsems[b])

    def process_chunk(c, b):
        pltpu.make_async_copy(
            table_hbm.at[pl.ds(0, CHUNK)], gath_v.at[b], gsems[b]).wait()
        pltpu.make_async_copy(
            x_hbm.at[wid].at[0], x_v.at[b], xsems[b]).wait()
        @pl.when(c >= 2)
        def _():
            pltpu.make_async_copy(
                out_v.at[b], out_hbm.at[wid].at[0], osems[b]).wait()

        for g in range(CHUNK // LANES):
            svec = lax.bitwise_and(
                idx_v[c, pl.ds(g * LANES, LANES)], MASK)
            for l in range(LANES):
                s = svec[l]
                j = g * LANES + l
                for d in range(EMBED // LANES):
                    dsl = pl.ds(d * LANES, LANES)
                    out_v[b, j, dsl] = gath_v[b, j, s, dsl] * x_v[b, j, dsl]

        pltpu.async_copy(out_v.at[b], out_hbm.at[wid].at[c], osems[b])

    issue_chunk(0, 0)

    def pair_body(i, carry):
        c0 = i * 2
        issue_chunk(c0 + 1, 1)
        process_chunk(c0, 0)

        @pl.when(c0 + 2 < NCH)
        def _():
            issue_chunk(c0 + 2, 0)

        process_chunk(c0 + 1, 1)
        return carry

    lax.fori_loop(0, NCH // 2, pair_body, 0)

    for b in range(2):
        pltpu.make_async_copy(
            out_v.at[b], out_hbm.at[wid].at[0], osems[b]).wait()


def kernel(x, id, table):
    idx = id.astype(jnp.int32).reshape(NW, NCH, CHUNK)
    x_r = x.reshape(NW, NCH, CHUNK, EMBED)
    table_t = table.reshape(table.shape[0] // TILE_R, TILE_R, EMBED)
    out = _sc_embed(x_r, idx, table_t)
    return out.reshape(BATCH, EMBED)
